# baseline (speedup 1.0000x reference)
import functools

import jax
import jax.numpy as jnp
from jax import lax
from jax.experimental import pallas as pl
from jax.experimental.pallas import tpu as pltpu
from jax.experimental.pallas import tpu_sc as plsc

N = 10000
E = 320000
R = 8
NB = 4
D = 128

NTILES = 16
NSC = 2
NW = NSC * NTILES
SCHUNK = 2000
GC = 128
NP2 = 10240
ACC2 = R * NP2
TSHARE = 20000
TREG = 20480
E2 = NTILES * TREG
TROWS = TREG // GC
NPASS = 8
CP = D // NPASS

_mesh = plsc.VectorSubcoreMesh(core_axis_name="c", subcore_axis_name="s")


@functools.partial(
    pl.kernel,
    out_type=(jax.ShapeDtypeStruct((E2,), jnp.int32),
              jax.ShapeDtypeStruct((E2,), jnp.int32),
              jax.ShapeDtypeStruct((R * N,), jnp.float32)),
    mesh=_mesh,
    scratch_types=[
        pltpu.VMEM((SCHUNK,), jnp.int32),
        pltpu.VMEM((SCHUNK,), jnp.int32),
        pltpu.VMEM((SCHUNK,), jnp.int32),
        pltpu.VMEM((SCHUNK,), jnp.int32),
        pltpu.VMEM((SCHUNK,), jnp.float32),
        pltpu.VMEM((SCHUNK,), jnp.int32),
        pltpu.VMEM((SCHUNK,), jnp.int32),
        pltpu.VMEM((480,), jnp.int32),
        pltpu.VMEM((480,), jnp.int32),
        pltpu.VMEM_SHARED((R * N,), jnp.float32),
    ],
)
def _prep_kernel(src_hbm, dst_hbm, et_hbm, sidx_hbm, gidx_hbm, cnt_hbm,
                 stage_s, stage_d, stage_t, seg_v, fbuf, sbuf, gbuf, spad,
                 gpad, cacc):
    c = lax.axis_index("c")
    s = lax.axis_index("s")
    z16 = jnp.zeros((16,), jnp.float32)
    one16 = jnp.full((16,), 1.0, jnp.float32)
    iota16 = lax.iota(jnp.int32, 16)

    def fill_zero(i, _):
        fbuf[pl.ds(i * 16, 16)] = z16
        return 0

    lax.fori_loop(0, SCHUNK // 16, fill_zero, 0)
    share = R * N // NTILES
    t0 = s * share
    pltpu.sync_copy(fbuf, cacc.at[pl.ds(t0, SCHUNK)])
    pltpu.sync_copy(fbuf, cacc.at[pl.ds(t0 + SCHUNK, SCHUNK)])
    pltpu.sync_copy(fbuf.at[pl.ds(0, share - 2 * SCHUNK)],
                    cacc.at[pl.ds(t0 + 2 * SCHUNK, share - 2 * SCHUNK)])
    plsc.subcore_barrier()

    def fill_one(i, _):
        fbuf[pl.ds(i * 16, 16)] = one16
        return 0

    lax.fori_loop(0, SCHUNK // 16, fill_one, 0)

    def count_chunk(j, _):
        ebase = s * TSHARE + j * SCHUNK
        pltpu.sync_copy(dst_hbm.at[pl.ds(ebase, SCHUNK)], stage_d)
        pltpu.sync_copy(et_hbm.at[pl.ds(ebase, SCHUNK)], stage_t)

        def vec(i, _):
            dst16 = stage_d[pl.ds(i * 16, 16)]
            typ16 = stage_t[pl.ds(i * 16, 16)]
            seg_v[pl.ds(i * 16, 16)] = dst16 * R + typ16
            return 0

        lax.fori_loop(0, SCHUNK // 16, vec, 0)
        pltpu.sync_copy(fbuf, cacc.at[seg_v], add=True)
        return 0

    lax.fori_loop(0, TSHARE // SCHUNK, count_chunk, 0)
    plsc.subcore_barrier()
    @pl.when(s < 8)
    def _():
        cbase = c * (R * N // 2) + s * 5000
        for q, ln in ((0, SCHUNK), (SCHUNK, SCHUNK), (2 * SCHUNK, 1000)):
            pltpu.sync_copy(cacc.at[pl.ds(cbase + q, ln)],
                            fbuf.at[pl.ds(0, ln)])
            pltpu.sync_copy(fbuf.at[pl.ds(0, ln)],
                            cnt_hbm.at[pl.ds(cbase + q, ln)])

    wid = c * NTILES + s
    obase = (wid // 2) * TREG + (wid % 2) * (E // NW)

    def emit_chunk(j, _):
        ebase = wid * (E // NW) + j * SCHUNK
        pltpu.sync_copy(src_hbm.at[pl.ds(ebase, SCHUNK)], stage_s)
        pltpu.sync_copy(dst_hbm.at[pl.ds(ebase, SCHUNK)], stage_d)
        pltpu.sync_copy(et_hbm.at[pl.ds(ebase, SCHUNK)], stage_t)

        def vseg(i, _):
            dst16 = stage_d[pl.ds(i * 16, 16)]
            typ16 = stage_t[pl.ds(i * 16, 16)]
            src16 = stage_s[pl.ds(i * 16, 16)]
            sbuf[pl.ds(i * 16, 16)] = typ16 * NP2 + dst16
            gbuf[pl.ds(i * 16, 16)] = src16
            return 0

        lax.fori_loop(0, SCHUNK // 16, vseg, 0)
        pltpu.sync_copy(sbuf, sidx_hbm.at[pl.ds(obase + j * SCHUNK, SCHUNK)])
        pltpu.sync_copy(gbuf, gidx_hbm.at[pl.ds(obase + j * SCHUNK, SCHUNK)])
        return 0

    lax.fori_loop(0, E // NW // SCHUNK, emit_chunk, 0)

    @pl.when(wid % 2 == 1)
    def _():
        def vpad(i, _):
            v = i * 16 + iota16
            spad[pl.ds(i * 16, 16)] = N + (v & 127)
            gpad[pl.ds(i * 16, 16)] = v & 8191
            return 0

        lax.fori_loop(0, 480 // 16, vpad, 0)
        pbase = (wid // 2) * TREG + E // NW * 2
        pltpu.sync_copy(spad, sidx_hbm.at[pl.ds(pbase, 480)])
        pltpu.sync_copy(gpad, gidx_hbm.at[pl.ds(pbase, 480)])


@functools.partial(
    pl.kernel,
    out_type=jax.ShapeDtypeStruct((NPASS, ACC2, CP), jnp.float32),
    mesh=_mesh,
    scratch_types=[
        pltpu.VMEM((TROWS, GC), jnp.int32),
        pltpu.VMEM((TROWS, GC), jnp.int32),
        pltpu.VMEM((GC, CP), jnp.float32),
        pltpu.VMEM((GC, CP), jnp.float32),
        pltpu.VMEM((64, CP), jnp.float32),
        pltpu.VMEM((64, CP), jnp.float32),
        pltpu.VMEM_SHARED((ACC2, CP), jnp.float32),
        pltpu.SemaphoreType.DMA,
        pltpu.SemaphoreType.DMA,
        pltpu.SemaphoreType.DMA,
    ],
    compiler_params=pltpu.CompilerParams(use_tc_tiling_on_sc=False),
)
def _agg_kernel(x2_hbm, gidx_hbm, sidx_hbm, out_hbm, gidxb, sidxb, rowsA,
                rowsB, zbuf, obuf, acc, semA, semB, semI):
    c = lax.axis_index("c")
    s = lax.axis_index("s")
    z16 = jnp.zeros((16,), jnp.float32)

    def zb(i, _):
        zbuf[i, pl.ds(0, 16)] = z16
        return 0

    lax.fori_loop(0, 64, zb, 0)

    ca = pltpu.async_copy(gidx_hbm.at[pl.ds(s * TROWS, TROWS)], gidxb, semI)
    pltpu.async_copy(sidx_hbm.at[pl.ds(s * TROWS, TROWS)], sidxb, semI)
    ca.wait()
    ca.wait()

    zshare = ACC2 // NTILES
    t0 = s * zshare
    oshare = zshare // 64

    for p in range(NPASS // 2):
        pass_id = c * (NPASS // 2) + p

        def zacc(i, _):
            pltpu.sync_copy(zbuf, acc.at[pl.ds(t0 + i * 64, 64)])
            return 0

        lax.fori_loop(0, oshare, zacc, 0)
        plsc.subcore_barrier()

        xp = x2_hbm.at[pass_id]

        def finish(rows, sem, sb_row):
            pltpu.make_async_copy(xp.at[pl.ds(0, GC)], rows, sem).wait()
            pltpu.sync_copy(rows, acc.at[sb_row], add=True)

        pltpu.async_copy(xp.at[gidxb.at[0]], rowsA, semA)

        def dchunk(it, _):
            k0 = it * 2
            pltpu.async_copy(xp.at[gidxb.at[k0 + 1]], rowsB, semB)
            finish(rowsA, semA, sidxb.at[k0])

            @pl.when(k0 + 2 < TROWS)
            def _():
                pltpu.async_copy(xp.at[gidxb.at[k0 + 2]], rowsA, semA)

            finish(rowsB, semB, sidxb.at[k0 + 1])
            return 0

        lax.fori_loop(0, TROWS // 2, dchunk, 0)
        plsc.subcore_barrier()

        def ocp(i, _):
            pltpu.sync_copy(acc.at[pl.ds(t0 + i * 64, 64)], obuf)
            pltpu.sync_copy(obuf,
                            out_hbm.at[pass_id, pl.ds(t0 + i * 64, 64)])
            return 0

        lax.fori_loop(0, oshare, ocp, 0)
        plsc.subcore_barrier()


BN = 400


def _bf(a):
    return a.astype(jnp.bfloat16).astype(jnp.float32)


def _combine_body(act, s0, s1, s2, s3, s4, s5, s6, s7, cnt_ref, h_ref,
                  basis_ref, comp_ref, root_ref, bias_ref, o_ref):
    sp = (s0, s1, s2, s3, s4, s5, s6, s7)
    hb = _bf(h_ref[...])
    acc = jnp.dot(hb, _bf(root_ref[...]),
                  preferred_element_type=jnp.float32)
    basis_b = _bf(basis_ref[...])
    for r in range(R):
        w_r = _bf(comp_ref[r, 0]) * basis_b[0]
        for b in range(1, NB):
            w_r = w_r + _bf(comp_ref[r, b]) * basis_b[b]
        s_r = jnp.concatenate([sp[p][r] for p in range(NPASS)], axis=1)
        mean_r = s_r / jnp.maximum(cnt_ref[:, r], 1.0)[:, None]
        acc = acc + jnp.dot(_bf(mean_r), _bf(w_r),
                            preferred_element_type=jnp.float32)
    acc = acc + bias_ref[...]
    if act == 0:
        o_ref[...] = jnp.maximum(acc, 0.0)
    else:
        mx = jnp.max(acc, axis=1, keepdims=True)
        e = jnp.exp(acc - mx)
        o_ref[...] = e / jnp.sum(e, axis=1, keepdims=True)


def _combine(act, sp3, cnt2, h, basis, comp, root, bias):
    sspec = pl.BlockSpec((R, BN, CP), lambda i: (0, i, 0))
    return pl.pallas_call(
        functools.partial(_combine_body, act),
        grid=(N // BN,),
        in_specs=[sspec] * NPASS + [
            pl.BlockSpec((BN, R), lambda i: (i, 0)),
            pl.BlockSpec((BN, D), lambda i: (i, 0)),
            pl.BlockSpec((NB, D, D), lambda i: (0, 0, 0)),
            pl.BlockSpec(memory_space=pltpu.SMEM),
            pl.BlockSpec((D, D), lambda i: (0, 0)),
            pl.BlockSpec((1, D), lambda i: (0, 0)),
        ],
        out_specs=pl.BlockSpec((BN, D), lambda i: (i, 0)),
        out_shape=jax.ShapeDtypeStruct((N, D), jnp.float32),
    )(*sp3, cnt2, h, basis, comp, root, bias.reshape(1, D))


def kernel(x, edge_index, edge_type, basis0, comp0, root0, bias0, basis1,
           comp1, root1, bias1, basis2, comp2, root2, bias2):
    src = edge_index[0]
    dst = edge_index[1]
    sidx, gidx, cnt = _prep_kernel(src, dst, edge_type)
    cnt2 = cnt.reshape(N, R)
    sidx2 = sidx.reshape(E2 // GC, GC)
    gidx2 = gidx.reshape(E2 // GC, GC)
    h = x
    layers = [(basis0, comp0, root0, bias0, 0),
              (basis1, comp1, root1, bias1, 0),
              (basis2, comp2, root2, bias2, 1)]
    for basis, comp, root, bias, act in layers:
        hp = jnp.pad(h, ((0, NP2 - N), (0, 0)))
        x2 = jnp.transpose(hp.reshape(NP2, NPASS, CP), (1, 0, 2))
        sums = _agg_kernel(x2, gidx2, sidx2)
        sp3 = [sums[p].reshape(R, NP2, CP) for p in range(NPASS)]
        h = _combine(act, sp3, cnt2, h, basis, comp, root, bias)
    return h


# speedup vs baseline: 1.1028x; 1.1028x over previous
import functools

import jax
import jax.numpy as jnp
from jax import lax
from jax.experimental import pallas as pl
from jax.experimental.pallas import tpu as pltpu
from jax.experimental.pallas import tpu_sc as plsc

N = 10000
E = 320000
R = 8
NB = 4
D = 128

NTILES = 16
NSC = 2
NW = NSC * NTILES
SCHUNK = 2000
GC = 128
NP2 = 10240
ACC2 = R * NP2
TSHARE = 20000
TREG = 20480
E2 = NTILES * TREG
TROWS = TREG // GC
NPASS = 8
CP = D // NPASS

_mesh = plsc.VectorSubcoreMesh(core_axis_name="c", subcore_axis_name="s")


@functools.partial(
    pl.kernel,
    out_type=(jax.ShapeDtypeStruct((E2,), jnp.int32),
              jax.ShapeDtypeStruct((E2,), jnp.int32),
              jax.ShapeDtypeStruct((R * N,), jnp.float32)),
    mesh=_mesh,
    scratch_types=[
        pltpu.VMEM((SCHUNK,), jnp.int32),
        pltpu.VMEM((SCHUNK,), jnp.int32),
        pltpu.VMEM((SCHUNK,), jnp.int32),
        pltpu.VMEM((SCHUNK,), jnp.int32),
        pltpu.VMEM((SCHUNK,), jnp.float32),
        pltpu.VMEM((SCHUNK,), jnp.int32),
        pltpu.VMEM((SCHUNK,), jnp.int32),
        pltpu.VMEM((480,), jnp.int32),
        pltpu.VMEM((480,), jnp.int32),
        pltpu.VMEM_SHARED((R * N,), jnp.float32),
    ],
)
def _prep_kernel(src_hbm, dst_hbm, et_hbm, sidx_hbm, gidx_hbm, cnt_hbm,
                 stage_s, stage_d, stage_t, seg_v, fbuf, sbuf, gbuf, spad,
                 gpad, cacc):
    c = lax.axis_index("c")
    s = lax.axis_index("s")
    z16 = jnp.zeros((16,), jnp.float32)
    one16 = jnp.full((16,), 1.0, jnp.float32)
    iota16 = lax.iota(jnp.int32, 16)

    def fill_zero(i, _):
        fbuf[pl.ds(i * 16, 16)] = z16
        return 0

    lax.fori_loop(0, SCHUNK // 16, fill_zero, 0)
    share = R * N // NTILES
    t0 = s * share
    pltpu.sync_copy(fbuf, cacc.at[pl.ds(t0, SCHUNK)])
    pltpu.sync_copy(fbuf, cacc.at[pl.ds(t0 + SCHUNK, SCHUNK)])
    pltpu.sync_copy(fbuf.at[pl.ds(0, share - 2 * SCHUNK)],
                    cacc.at[pl.ds(t0 + 2 * SCHUNK, share - 2 * SCHUNK)])
    plsc.subcore_barrier()

    def fill_one(i, _):
        fbuf[pl.ds(i * 16, 16)] = one16
        return 0

    lax.fori_loop(0, SCHUNK // 16, fill_one, 0)

    def count_chunk(j, _):
        ebase = s * TSHARE + j * SCHUNK
        pltpu.sync_copy(dst_hbm.at[pl.ds(ebase, SCHUNK)], stage_d)
        pltpu.sync_copy(et_hbm.at[pl.ds(ebase, SCHUNK)], stage_t)

        def vec(i, _):
            dst16 = stage_d[pl.ds(i * 16, 16)]
            typ16 = stage_t[pl.ds(i * 16, 16)]
            seg_v[pl.ds(i * 16, 16)] = dst16 * R + typ16
            return 0

        lax.fori_loop(0, SCHUNK // 16, vec, 0)
        pltpu.sync_copy(fbuf, cacc.at[seg_v], add=True)
        return 0

    lax.fori_loop(0, TSHARE // SCHUNK, count_chunk, 0)
    plsc.subcore_barrier()
    @pl.when(s < 8)
    def _():
        cbase = c * (R * N // 2) + s * 5000
        for q, ln in ((0, SCHUNK), (SCHUNK, SCHUNK), (2 * SCHUNK, 1000)):
            pltpu.sync_copy(cacc.at[pl.ds(cbase + q, ln)],
                            fbuf.at[pl.ds(0, ln)])
            pltpu.sync_copy(fbuf.at[pl.ds(0, ln)],
                            cnt_hbm.at[pl.ds(cbase + q, ln)])

    wid = c * NTILES + s
    obase = (wid // 2) * TREG + (wid % 2) * (E // NW)

    def emit_chunk(j, _):
        ebase = wid * (E // NW) + j * SCHUNK
        pltpu.sync_copy(src_hbm.at[pl.ds(ebase, SCHUNK)], stage_s)
        pltpu.sync_copy(dst_hbm.at[pl.ds(ebase, SCHUNK)], stage_d)
        pltpu.sync_copy(et_hbm.at[pl.ds(ebase, SCHUNK)], stage_t)

        def vseg(i, _):
            dst16 = stage_d[pl.ds(i * 16, 16)]
            typ16 = stage_t[pl.ds(i * 16, 16)]
            src16 = stage_s[pl.ds(i * 16, 16)]
            sbuf[pl.ds(i * 16, 16)] = typ16 * NP2 + dst16
            gbuf[pl.ds(i * 16, 16)] = src16
            return 0

        lax.fori_loop(0, SCHUNK // 16, vseg, 0)
        pltpu.sync_copy(sbuf, sidx_hbm.at[pl.ds(obase + j * SCHUNK, SCHUNK)])
        pltpu.sync_copy(gbuf, gidx_hbm.at[pl.ds(obase + j * SCHUNK, SCHUNK)])
        return 0

    lax.fori_loop(0, E // NW // SCHUNK, emit_chunk, 0)

    @pl.when(wid % 2 == 1)
    def _():
        def vpad(i, _):
            v = i * 16 + iota16
            spad[pl.ds(i * 16, 16)] = N + (v & 127)
            gpad[pl.ds(i * 16, 16)] = v & 8191
            return 0

        lax.fori_loop(0, 480 // 16, vpad, 0)
        pbase = (wid // 2) * TREG + E // NW * 2
        pltpu.sync_copy(spad, sidx_hbm.at[pl.ds(pbase, 480)])
        pltpu.sync_copy(gpad, gidx_hbm.at[pl.ds(pbase, 480)])


@functools.partial(
    pl.kernel,
    out_type=jax.ShapeDtypeStruct((NPASS, ACC2, CP), jnp.float32),
    mesh=_mesh,
    scratch_types=[
        pltpu.VMEM((GC,), jnp.int32),
        pltpu.VMEM((GC,), jnp.int32),
        pltpu.VMEM((GC,), jnp.int32),
        pltpu.VMEM((GC,), jnp.int32),
        pltpu.VMEM((GC,), jnp.int32),
        pltpu.VMEM((GC,), jnp.int32),
        pltpu.VMEM((GC,), jnp.int32),
        pltpu.VMEM((GC,), jnp.int32),
        pltpu.VMEM((GC, CP), jnp.float32),
        pltpu.VMEM((GC, CP), jnp.float32),
        pltpu.VMEM((GC, CP), jnp.float32),
        pltpu.VMEM((GC, CP), jnp.float32),
        pltpu.VMEM((512, CP), jnp.float32),
        pltpu.VMEM((512, CP), jnp.float32),
        pltpu.VMEM_SHARED((ACC2, CP), jnp.float32),
    ] + [pltpu.SemaphoreType.DMA] * 12,
    compiler_params=pltpu.CompilerParams(use_tc_tiling_on_sc=False),
)
def _agg_kernel(x2_hbm, gidx_hbm, sidx_hbm, out_hbm, gi0, gi1, gi2, gi3,
                si0, si1, si2, si3, rw0, rw1, rw2, rw3, zbuf, obuf, acc,
                sI0, sI1, sI2, sI3, sG0, sG1, sG2, sG3, sS0, sS1, sS2, sS3):
    GI = (gi0, gi1, gi2, gi3)
    SI = (si0, si1, si2, si3)
    RW = (rw0, rw1, rw2, rw3)
    SEMI = (sI0, sI1, sI2, sI3)
    SEMG = (sG0, sG1, sG2, sG3)
    SEMS = (sS0, sS1, sS2, sS3)
    c = lax.axis_index("c")
    s = lax.axis_index("s")
    z16 = jnp.zeros((16,), jnp.float32)

    def zb(i, _):
        zbuf[i, pl.ds(0, 16)] = z16
        return 0

    lax.fori_loop(0, 512, zb, 0)

    zshare = ACC2 // NTILES
    t0 = s * zshare
    oshare = zshare // 512

    def stage_idx(k, j):
        e0 = s * TREG + k * GC
        pltpu.async_copy(gidx_hbm.at[pl.ds(e0, GC)], GI[j], SEMI[j])
        pltpu.async_copy(sidx_hbm.at[pl.ds(e0, GC)], SI[j], SEMI[j])

    def wait_idx(j):
        d = pltpu.make_async_copy(gidx_hbm.at[pl.ds(0, GC)], GI[j], SEMI[j])
        d.wait()
        d.wait()

    for p in range(NPASS // 2):
        pass_id = c * (NPASS // 2) + p

        def zacc(i, _):
            pltpu.sync_copy(zbuf, acc.at[pl.ds(t0 + i * 512, 512)])
            return 0

        lax.fori_loop(0, oshare, zacc, 0)
        plsc.subcore_barrier()

        xp = x2_hbm.at[pass_id]

        stage_idx(0, 0)
        stage_idx(1, 1)
        stage_idx(2, 2)
        wait_idx(0)
        pltpu.async_copy(xp.at[GI[0]], RW[0], SEMG[0])

        def qblock(it, _):
            for jj in range(4):
                k = it * 4 + jj
                jn = (jj + 1) % 4
                jp = (jj + 3) % 4

                @pl.when(k + 3 < TROWS)
                def _():
                    stage_idx(k + 3, jp)

                @pl.when(k + 1 < TROWS)
                def _():
                    wait_idx(jn)

                    @pl.when(k >= 3)
                    def _():
                        pltpu.make_async_copy(RW[jn], acc.at[SI[jn]],
                                              SEMS[jn]).wait()

                    pltpu.async_copy(xp.at[GI[jn]], RW[jn], SEMG[jn])

                pltpu.make_async_copy(xp.at[pl.ds(0, GC)], RW[jj],
                                      SEMG[jj]).wait()
                pltpu.async_copy(RW[jj], acc.at[SI[jj]], SEMS[jj], add=True)
            return 0

        lax.fori_loop(0, TROWS // 4, qblock, 0)
        for jj in range(4):
            pltpu.make_async_copy(RW[jj], acc.at[SI[jj]], SEMS[jj]).wait()
        plsc.subcore_barrier()

        def ocp(i, _):
            pltpu.sync_copy(acc.at[pl.ds(t0 + i * 512, 512)], obuf)
            pltpu.sync_copy(obuf,
                            out_hbm.at[pass_id, pl.ds(t0 + i * 512, 512)])
            return 0

        lax.fori_loop(0, oshare, ocp, 0)
        plsc.subcore_barrier()


BN = 400


def _bf(a):
    return a.astype(jnp.bfloat16).astype(jnp.float32)


def _combine_body(act, s0, s1, s2, s3, s4, s5, s6, s7, cnt_ref, h_ref,
                  basis_ref, comp_ref, root_ref, bias_ref, o_ref):
    sp = (s0, s1, s2, s3, s4, s5, s6, s7)
    hb = _bf(h_ref[...])
    acc = jnp.dot(hb, _bf(root_ref[...]),
                  preferred_element_type=jnp.float32)
    basis_b = _bf(basis_ref[...])
    for r in range(R):
        w_r = _bf(comp_ref[r, 0]) * basis_b[0]
        for b in range(1, NB):
            w_r = w_r + _bf(comp_ref[r, b]) * basis_b[b]
        s_r = jnp.concatenate([sp[p][r] for p in range(NPASS)], axis=1)
        mean_r = s_r / jnp.maximum(cnt_ref[:, r], 1.0)[:, None]
        acc = acc + jnp.dot(_bf(mean_r), _bf(w_r),
                            preferred_element_type=jnp.float32)
    acc = acc + bias_ref[...]
    if act == 0:
        res = jnp.maximum(acc, 0.0)
    else:
        mx = jnp.max(acc, axis=1, keepdims=True)
        e = jnp.exp(acc - mx)
        res = e / jnp.sum(e, axis=1, keepdims=True)
    o_ref[...] = res


def _combine(act, sp3, cnt2, h, basis, comp, root, bias):
    sspec = pl.BlockSpec((R, BN, CP), lambda i: (0, i, 0))
    return pl.pallas_call(
        functools.partial(_combine_body, act),
        grid=(N // BN,),
        in_specs=[sspec] * NPASS + [
            pl.BlockSpec((BN, R), lambda i: (i, 0)),
            pl.BlockSpec((BN, D), lambda i: (i, 0)),
            pl.BlockSpec((NB, D, D), lambda i: (0, 0, 0)),
            pl.BlockSpec(memory_space=pltpu.SMEM),
            pl.BlockSpec((D, D), lambda i: (0, 0)),
            pl.BlockSpec((1, D), lambda i: (0, 0)),
        ],
        out_specs=pl.BlockSpec((BN, D), lambda i: (i, 0)),
        out_shape=jax.ShapeDtypeStruct((N, D), jnp.float32),
    )(*sp3, cnt2, h, basis, comp, root, bias.reshape(1, D))


def kernel(x, edge_index, edge_type, basis0, comp0, root0, bias0, basis1,
           comp1, root1, bias1, basis2, comp2, root2, bias2):
    src = edge_index[0]
    dst = edge_index[1]
    sidx, gidx, cnt = _prep_kernel(src, dst, edge_type)
    cnt2 = cnt.reshape(N, R)
    h = x
    layers = [(basis0, comp0, root0, bias0, 0),
              (basis1, comp1, root1, bias1, 0),
              (basis2, comp2, root2, bias2, 1)]
    for basis, comp, root, bias, act in layers:
        x2 = jnp.transpose(h.reshape(N, NPASS, CP), (1, 0, 2))
        sums = _agg_kernel(x2, gidx, sidx)
        sp3 = [sums[p].reshape(R, NP2, CP) for p in range(NPASS)]
        h = _combine(act, sp3, cnt2, h, basis, comp, root, bias)
    return h


# speedup vs baseline: 1.2758x; 1.1569x over previous
import functools

import jax
import jax.numpy as jnp
from jax import lax
from jax.experimental import pallas as pl
from jax.experimental.pallas import tpu as pltpu
from jax.experimental.pallas import tpu_sc as plsc

N = 10000
E = 320000
R = 8
NB = 4
D = 128

NTILES = 16
NSC = 2
NW = NSC * NTILES
SCHUNK = 2000
GC = 128
NP2 = 10240
ACC2 = R * NP2
TSHARE = 20000
TREG = 20480
E2 = NTILES * TREG
TROWS = TREG // GC
NPASS = 8
CP = D // NPASS

_mesh = plsc.VectorSubcoreMesh(core_axis_name="c", subcore_axis_name="s")


@functools.partial(
    pl.kernel,
    out_type=(jax.ShapeDtypeStruct((E2,), jnp.int32),
              jax.ShapeDtypeStruct((E2,), jnp.int32),
              jax.ShapeDtypeStruct((R * N,), jnp.float32)),
    mesh=_mesh,
    scratch_types=[
        pltpu.VMEM((SCHUNK,), jnp.int32),
        pltpu.VMEM((SCHUNK,), jnp.int32),
        pltpu.VMEM((SCHUNK,), jnp.int32),
        pltpu.VMEM((SCHUNK,), jnp.int32),
        pltpu.VMEM((SCHUNK,), jnp.float32),
        pltpu.VMEM((SCHUNK,), jnp.int32),
        pltpu.VMEM((SCHUNK,), jnp.int32),
        pltpu.VMEM((480,), jnp.int32),
        pltpu.VMEM((480,), jnp.int32),
        pltpu.VMEM_SHARED((R * N,), jnp.float32),
    ],
)
def _prep_kernel(src_hbm, dst_hbm, et_hbm, sidx_hbm, gidx_hbm, cnt_hbm,
                 stage_s, stage_d, stage_t, seg_v, fbuf, sbuf, gbuf, spad,
                 gpad, cacc):
    c = lax.axis_index("c")
    s = lax.axis_index("s")
    z16 = jnp.zeros((16,), jnp.float32)
    one16 = jnp.full((16,), 1.0, jnp.float32)
    iota16 = lax.iota(jnp.int32, 16)

    def fill_zero(i, _):
        fbuf[pl.ds(i * 16, 16)] = z16
        return 0

    lax.fori_loop(0, SCHUNK // 16, fill_zero, 0)
    share = R * N // NTILES
    t0 = s * share
    pltpu.sync_copy(fbuf, cacc.at[pl.ds(t0, SCHUNK)])
    pltpu.sync_copy(fbuf, cacc.at[pl.ds(t0 + SCHUNK, SCHUNK)])
    pltpu.sync_copy(fbuf.at[pl.ds(0, share - 2 * SCHUNK)],
                    cacc.at[pl.ds(t0 + 2 * SCHUNK, share - 2 * SCHUNK)])
    plsc.subcore_barrier()

    def fill_one(i, _):
        fbuf[pl.ds(i * 16, 16)] = one16
        return 0

    lax.fori_loop(0, SCHUNK // 16, fill_one, 0)

    def count_chunk(j, _):
        ebase = s * TSHARE + j * SCHUNK
        pltpu.sync_copy(dst_hbm.at[pl.ds(ebase, SCHUNK)], stage_d)
        pltpu.sync_copy(et_hbm.at[pl.ds(ebase, SCHUNK)], stage_t)

        def vec(i, _):
            dst16 = stage_d[pl.ds(i * 16, 16)]
            typ16 = stage_t[pl.ds(i * 16, 16)]
            seg_v[pl.ds(i * 16, 16)] = dst16 * R + typ16
            return 0

        lax.fori_loop(0, SCHUNK // 16, vec, 0)
        pltpu.sync_copy(fbuf, cacc.at[seg_v], add=True)
        return 0

    lax.fori_loop(0, TSHARE // SCHUNK, count_chunk, 0)
    plsc.subcore_barrier()
    @pl.when(s < 8)
    def _():
        cbase = c * (R * N // 2) + s * 5000
        for q, ln in ((0, SCHUNK), (SCHUNK, SCHUNK), (2 * SCHUNK, 1000)):
            pltpu.sync_copy(cacc.at[pl.ds(cbase + q, ln)],
                            fbuf.at[pl.ds(0, ln)])
            pltpu.sync_copy(fbuf.at[pl.ds(0, ln)],
                            cnt_hbm.at[pl.ds(cbase + q, ln)])

    wid = c * NTILES + s
    obase = (wid // 2) * TREG + (wid % 2) * (E // NW)

    def emit_chunk(j, _):
        ebase = wid * (E // NW) + j * SCHUNK
        pltpu.sync_copy(src_hbm.at[pl.ds(ebase, SCHUNK)], stage_s)
        pltpu.sync_copy(dst_hbm.at[pl.ds(ebase, SCHUNK)], stage_d)
        pltpu.sync_copy(et_hbm.at[pl.ds(ebase, SCHUNK)], stage_t)

        def vseg(i, _):
            dst16 = stage_d[pl.ds(i * 16, 16)]
            typ16 = stage_t[pl.ds(i * 16, 16)]
            src16 = stage_s[pl.ds(i * 16, 16)]
            sbuf[pl.ds(i * 16, 16)] = typ16 * NP2 + dst16
            gbuf[pl.ds(i * 16, 16)] = src16
            return 0

        lax.fori_loop(0, SCHUNK // 16, vseg, 0)
        pltpu.sync_copy(sbuf, sidx_hbm.at[pl.ds(obase + j * SCHUNK, SCHUNK)])
        pltpu.sync_copy(gbuf, gidx_hbm.at[pl.ds(obase + j * SCHUNK, SCHUNK)])
        return 0

    lax.fori_loop(0, E // NW // SCHUNK, emit_chunk, 0)

    @pl.when(wid % 2 == 1)
    def _():
        def vpad(i, _):
            v = i * 16 + iota16
            spad[pl.ds(i * 16, 16)] = N + (v & 127)
            gpad[pl.ds(i * 16, 16)] = v & 8191
            return 0

        lax.fori_loop(0, 480 // 16, vpad, 0)
        pbase = (wid // 2) * TREG + E // NW * 2
        pltpu.sync_copy(spad, sidx_hbm.at[pl.ds(pbase, 480)])
        pltpu.sync_copy(gpad, gidx_hbm.at[pl.ds(pbase, 480)])


@functools.partial(
    pl.kernel,
    out_type=jax.ShapeDtypeStruct((NPASS, ACC2, CP), jnp.float32),
    mesh=_mesh,
    scratch_types=[
        pltpu.VMEM((GC,), jnp.int32),
        pltpu.VMEM((GC,), jnp.int32),
        pltpu.VMEM((GC,), jnp.int32),
        pltpu.VMEM((GC,), jnp.int32),
        pltpu.VMEM((GC,), jnp.int32),
        pltpu.VMEM((GC,), jnp.int32),
        pltpu.VMEM((GC,), jnp.int32),
        pltpu.VMEM((GC,), jnp.int32),
        pltpu.VMEM((GC, CP), jnp.float32),
        pltpu.VMEM((GC, CP), jnp.float32),
        pltpu.VMEM((GC, CP), jnp.float32),
        pltpu.VMEM((GC, CP), jnp.float32),
        pltpu.VMEM((512, CP), jnp.float32),
        pltpu.VMEM((512, CP), jnp.float32),
        pltpu.VMEM_SHARED((ACC2, CP), jnp.float32),
        pltpu.VMEM_SHARED((NP2, CP), jnp.float32),
    ] + [pltpu.SemaphoreType.DMA] * 12,
    compiler_params=pltpu.CompilerParams(use_tc_tiling_on_sc=False),
)
def _agg_kernel(x2_hbm, gidx_hbm, sidx_hbm, out_hbm, gi0, gi1, gi2, gi3,
                si0, si1, si2, si3, rw0, rw1, rw2, rw3, zbuf, obuf, acc, xs,
                sI0, sI1, sI2, sI3, sG0, sG1, sG2, sG3, sS0, sS1, sS2, sS3):
    GI = (gi0, gi1, gi2, gi3)
    SI = (si0, si1, si2, si3)
    RW = (rw0, rw1, rw2, rw3)
    SEMI = (sI0, sI1, sI2, sI3)
    SEMG = (sG0, sG1, sG2, sG3)
    SEMS = (sS0, sS1, sS2, sS3)
    c = lax.axis_index("c")
    s = lax.axis_index("s")
    z16 = jnp.zeros((16,), jnp.float32)

    def zb(i, _):
        zbuf[i, pl.ds(0, 16)] = z16
        return 0

    lax.fori_loop(0, 512, zb, 0)

    zshare = ACC2 // NTILES
    t0 = s * zshare
    oshare = zshare // 512

    def stage_idx(k, j):
        e0 = s * TREG + k * GC
        pltpu.async_copy(gidx_hbm.at[pl.ds(e0, GC)], GI[j], SEMI[j])
        pltpu.async_copy(sidx_hbm.at[pl.ds(e0, GC)], SI[j], SEMI[j])

    def wait_idx(j):
        d = pltpu.make_async_copy(gidx_hbm.at[pl.ds(0, GC)], GI[j], SEMI[j])
        d.wait()
        d.wait()

    for p in range(NPASS // 2):
        pass_id = c * (NPASS // 2) + p

        def zacc(i, _):
            pltpu.sync_copy(zbuf, acc.at[pl.ds(t0 + i * 512, 512)])
            return 0

        lax.fori_loop(0, oshare, zacc, 0)

        xp = x2_hbm.at[pass_id]
        x0 = s * (N // NTILES)
        pltpu.sync_copy(xp.at[pl.ds(x0, 512)], obuf)
        pltpu.sync_copy(obuf, xs.at[pl.ds(x0, 512)])
        pltpu.sync_copy(xp.at[pl.ds(x0 + 512, 113)], obuf.at[pl.ds(0, 113)])
        pltpu.sync_copy(obuf.at[pl.ds(0, 113)], xs.at[pl.ds(x0 + 512, 113)])
        plsc.subcore_barrier()

        stage_idx(0, 0)
        stage_idx(1, 1)
        stage_idx(2, 2)
        wait_idx(0)
        pltpu.async_copy(xs.at[GI[0]], RW[0], SEMG[0])

        def qblock(it, _):
            for jj in range(4):
                k = it * 4 + jj
                jn = (jj + 1) % 4
                jp = (jj + 3) % 4

                @pl.when(k + 3 < TROWS)
                def _():
                    stage_idx(k + 3, jp)

                @pl.when(k + 1 < TROWS)
                def _():
                    wait_idx(jn)

                    @pl.when(k >= 3)
                    def _():
                        pltpu.make_async_copy(RW[jn], acc.at[SI[jn]],
                                              SEMS[jn]).wait()

                    pltpu.async_copy(xs.at[GI[jn]], RW[jn], SEMG[jn])

                pltpu.make_async_copy(xp.at[pl.ds(0, GC)], RW[jj],
                                      SEMG[jj]).wait()
                pltpu.async_copy(RW[jj], acc.at[SI[jj]], SEMS[jj], add=True)
            return 0

        lax.fori_loop(0, TROWS // 4, qblock, 0)
        for jj in range(4):
            pltpu.make_async_copy(RW[jj], acc.at[SI[jj]], SEMS[jj]).wait()
        plsc.subcore_barrier()

        def ocp(i, _):
            pltpu.sync_copy(acc.at[pl.ds(t0 + i * 512, 512)], obuf)
            pltpu.sync_copy(obuf,
                            out_hbm.at[pass_id, pl.ds(t0 + i * 512, 512)])
            return 0

        lax.fori_loop(0, oshare, ocp, 0)
        plsc.subcore_barrier()


BN = 400


def _bf(a):
    return a.astype(jnp.bfloat16).astype(jnp.float32)


def _combine_body(act, s0, s1, s2, s3, s4, s5, s6, s7, cnt_ref, h_ref,
                  basis_ref, comp_ref, root_ref, bias_ref, o_ref):
    sp = (s0, s1, s2, s3, s4, s5, s6, s7)
    hb = _bf(h_ref[...])
    acc = jnp.dot(hb, _bf(root_ref[...]),
                  preferred_element_type=jnp.float32)
    basis_b = _bf(basis_ref[...])
    for r in range(R):
        w_r = _bf(comp_ref[r, 0]) * basis_b[0]
        for b in range(1, NB):
            w_r = w_r + _bf(comp_ref[r, b]) * basis_b[b]
        s_r = jnp.concatenate([sp[p][r] for p in range(NPASS)], axis=1)
        mean_r = s_r / jnp.maximum(cnt_ref[:, r], 1.0)[:, None]
        acc = acc + jnp.dot(_bf(mean_r), _bf(w_r),
                            preferred_element_type=jnp.float32)
    acc = acc + bias_ref[...]
    if act == 0:
        res = jnp.maximum(acc, 0.0)
    else:
        mx = jnp.max(acc, axis=1, keepdims=True)
        e = jnp.exp(acc - mx)
        res = e / jnp.sum(e, axis=1, keepdims=True)
    o_ref[...] = res


def _combine(act, sp3, cnt2, h, basis, comp, root, bias):
    sspec = pl.BlockSpec((R, BN, CP), lambda i: (0, i, 0))
    return pl.pallas_call(
        functools.partial(_combine_body, act),
        grid=(N // BN,),
        in_specs=[sspec] * NPASS + [
            pl.BlockSpec((BN, R), lambda i: (i, 0)),
            pl.BlockSpec((BN, D), lambda i: (i, 0)),
            pl.BlockSpec((NB, D, D), lambda i: (0, 0, 0)),
            pl.BlockSpec(memory_space=pltpu.SMEM),
            pl.BlockSpec((D, D), lambda i: (0, 0)),
            pl.BlockSpec((1, D), lambda i: (0, 0)),
        ],
        out_specs=pl.BlockSpec((BN, D), lambda i: (i, 0)),
        out_shape=jax.ShapeDtypeStruct((N, D), jnp.float32),
    )(*sp3, cnt2, h, basis, comp, root, bias.reshape(1, D))


def kernel(x, edge_index, edge_type, basis0, comp0, root0, bias0, basis1,
           comp1, root1, bias1, basis2, comp2, root2, bias2):
    src = edge_index[0]
    dst = edge_index[1]
    sidx, gidx, cnt = _prep_kernel(src, dst, edge_type)
    cnt2 = cnt.reshape(N, R)
    h = x
    layers = [(basis0, comp0, root0, bias0, 0),
              (basis1, comp1, root1, bias1, 0),
              (basis2, comp2, root2, bias2, 1)]
    for basis, comp, root, bias, act in layers:
        x2 = jnp.transpose(h.reshape(N, NPASS, CP), (1, 0, 2))
        sums = _agg_kernel(x2, gidx, sidx)
        sp3 = [sums[p].reshape(R, NP2, CP) for p in range(NPASS)]
        h = _combine(act, sp3, cnt2, h, basis, comp, root, bias)
    return h


# speedup vs baseline: 3.3604x; 2.6340x over previous
import functools

import jax
import jax.numpy as jnp
from jax import lax
from jax.experimental import pallas as pl
from jax.experimental.pallas import tpu as pltpu
from jax.experimental.pallas import tpu_sc as plsc

N = 10000
E = 320000
R = 8
NB = 4
D = 128

NTILES = 16
NSC = 2
NW = NSC * NTILES
SCHUNK = 2000
GC = 128
NP2 = 10240
ACC2 = R * NP2
TSHARE = 20000
TREG = 20480
E2 = NTILES * TREG
TROWS = TREG // GC
NPASS = 8
CP = D // NPASS

_mesh = plsc.VectorSubcoreMesh(core_axis_name="c", subcore_axis_name="s")


@functools.partial(
    pl.kernel,
    out_type=(jax.ShapeDtypeStruct((E2,), jnp.int32),
              jax.ShapeDtypeStruct((E2,), jnp.int32),
              jax.ShapeDtypeStruct((R * N,), jnp.float32)),
    mesh=_mesh,
    scratch_types=[
        pltpu.VMEM((SCHUNK,), jnp.int32),
        pltpu.VMEM((SCHUNK,), jnp.int32),
        pltpu.VMEM((SCHUNK,), jnp.int32),
        pltpu.VMEM((SCHUNK,), jnp.int32),
        pltpu.VMEM((SCHUNK,), jnp.float32),
        pltpu.VMEM((SCHUNK,), jnp.int32),
        pltpu.VMEM((SCHUNK,), jnp.int32),
        pltpu.VMEM((480,), jnp.int32),
        pltpu.VMEM((480,), jnp.int32),
        pltpu.VMEM_SHARED((R * N,), jnp.float32),
    ],
)
def _prep_kernel(src_hbm, dst_hbm, et_hbm, sidx_hbm, gidx_hbm, cnt_hbm,
                 stage_s, stage_d, stage_t, seg_v, fbuf, sbuf, gbuf, spad,
                 gpad, cacc):
    c = lax.axis_index("c")
    s = lax.axis_index("s")
    z16 = jnp.zeros((16,), jnp.float32)
    one16 = jnp.full((16,), 1.0, jnp.float32)
    iota16 = lax.iota(jnp.int32, 16)

    def fill_zero(i, _):
        fbuf[pl.ds(i * 16, 16)] = z16
        return 0

    lax.fori_loop(0, SCHUNK // 16, fill_zero, 0)
    share = R * N // NTILES
    t0 = s * share
    pltpu.sync_copy(fbuf, cacc.at[pl.ds(t0, SCHUNK)])
    pltpu.sync_copy(fbuf, cacc.at[pl.ds(t0 + SCHUNK, SCHUNK)])
    pltpu.sync_copy(fbuf.at[pl.ds(0, share - 2 * SCHUNK)],
                    cacc.at[pl.ds(t0 + 2 * SCHUNK, share - 2 * SCHUNK)])
    plsc.subcore_barrier()

    def fill_one(i, _):
        fbuf[pl.ds(i * 16, 16)] = one16
        return 0

    lax.fori_loop(0, SCHUNK // 16, fill_one, 0)

    def count_chunk(j, _):
        ebase = s * TSHARE + j * SCHUNK
        pltpu.sync_copy(dst_hbm.at[pl.ds(ebase, SCHUNK)], stage_d)
        pltpu.sync_copy(et_hbm.at[pl.ds(ebase, SCHUNK)], stage_t)

        def vec(i, _):
            dst16 = stage_d[pl.ds(i * 16, 16)]
            typ16 = stage_t[pl.ds(i * 16, 16)]
            seg_v[pl.ds(i * 16, 16)] = dst16 * R + typ16
            return 0

        lax.fori_loop(0, SCHUNK // 16, vec, 0)
        pltpu.sync_copy(fbuf, cacc.at[seg_v], add=True)
        return 0

    lax.fori_loop(0, TSHARE // SCHUNK, count_chunk, 0)
    plsc.subcore_barrier()
    @pl.when(s < 8)
    def _():
        cbase = c * (R * N // 2) + s * 5000
        for q, ln in ((0, SCHUNK), (SCHUNK, SCHUNK), (2 * SCHUNK, 1000)):
            pltpu.sync_copy(cacc.at[pl.ds(cbase + q, ln)],
                            fbuf.at[pl.ds(0, ln)])
            pltpu.sync_copy(fbuf.at[pl.ds(0, ln)],
                            cnt_hbm.at[pl.ds(cbase + q, ln)])

    wid = c * NTILES + s
    obase = (wid // 2) * TREG + (wid % 2) * (E // NW)

    def emit_chunk(j, _):
        ebase = wid * (E // NW) + j * SCHUNK
        pltpu.sync_copy(src_hbm.at[pl.ds(ebase, SCHUNK)], stage_s)
        pltpu.sync_copy(dst_hbm.at[pl.ds(ebase, SCHUNK)], stage_d)
        pltpu.sync_copy(et_hbm.at[pl.ds(ebase, SCHUNK)], stage_t)

        def vseg(i, _):
            dst16 = stage_d[pl.ds(i * 16, 16)]
            typ16 = stage_t[pl.ds(i * 16, 16)]
            src16 = stage_s[pl.ds(i * 16, 16)]
            sbuf[pl.ds(i * 16, 16)] = typ16 * NP2 + dst16
            gbuf[pl.ds(i * 16, 16)] = src16
            return 0

        lax.fori_loop(0, SCHUNK // 16, vseg, 0)
        pltpu.sync_copy(sbuf, sidx_hbm.at[pl.ds(obase + j * SCHUNK, SCHUNK)])
        pltpu.sync_copy(gbuf, gidx_hbm.at[pl.ds(obase + j * SCHUNK, SCHUNK)])
        return 0

    lax.fori_loop(0, E // NW // SCHUNK, emit_chunk, 0)

    @pl.when(wid % 2 == 1)
    def _():
        def vpad(i, _):
            v = i * 16 + iota16
            spad[pl.ds(i * 16, 16)] = N + (v & 127)
            gpad[pl.ds(i * 16, 16)] = v & 8191
            return 0

        lax.fori_loop(0, 480 // 16, vpad, 0)
        pbase = (wid // 2) * TREG + E // NW * 2
        pltpu.sync_copy(spad, sidx_hbm.at[pl.ds(pbase, 480)])
        pltpu.sync_copy(gpad, gidx_hbm.at[pl.ds(pbase, 480)])


@functools.partial(
    pl.kernel,
    out_type=jax.ShapeDtypeStruct((ACC2, D), jnp.float32),
    mesh=_mesh,
    scratch_types=[
        pltpu.VMEM((GC,), jnp.int32),
        pltpu.VMEM((GC,), jnp.int32),
        pltpu.VMEM((GC,), jnp.int32),
        pltpu.VMEM((GC,), jnp.int32),
        pltpu.VMEM((GC,), jnp.int32),
        pltpu.VMEM((GC,), jnp.int32),
        pltpu.VMEM((GC,), jnp.int32),
        pltpu.VMEM((GC,), jnp.int32),
        pltpu.VMEM((GC, CP), jnp.float32),
        pltpu.VMEM((GC, CP), jnp.float32),
        pltpu.VMEM((GC, CP), jnp.float32),
        pltpu.VMEM((GC, CP), jnp.float32),
        pltpu.VMEM((512, CP), jnp.float32),
        pltpu.VMEM((512, CP), jnp.float32),
        pltpu.VMEM_SHARED((ACC2, CP), jnp.float32),
        pltpu.VMEM_SHARED((NP2, CP), jnp.float32),
    ] + [pltpu.SemaphoreType.DMA] * 12,
    compiler_params=pltpu.CompilerParams(use_tc_tiling_on_sc=False),
)
def _agg_kernel(x2_hbm, gidx_hbm, sidx_hbm, out_hbm, gi0, gi1, gi2, gi3,
                si0, si1, si2, si3, rw0, rw1, rw2, rw3, zbuf, obuf, acc, xs,
                sI0, sI1, sI2, sI3, sG0, sG1, sG2, sG3, sS0, sS1, sS2, sS3):
    GI = (gi0, gi1, gi2, gi3)
    SI = (si0, si1, si2, si3)
    RW = (rw0, rw1, rw2, rw3)
    SEMI = (sI0, sI1, sI2, sI3)
    SEMG = (sG0, sG1, sG2, sG3)
    SEMS = (sS0, sS1, sS2, sS3)
    c = lax.axis_index("c")
    s = lax.axis_index("s")
    z16 = jnp.zeros((16,), jnp.float32)

    def zb(i, _):
        zbuf[i, pl.ds(0, 16)] = z16
        return 0

    lax.fori_loop(0, 512, zb, 0)

    zshare = ACC2 // NTILES
    t0 = s * zshare
    oshare = zshare // 512

    def stage_idx(k, j):
        e0 = s * TREG + k * GC
        pltpu.async_copy(gidx_hbm.at[pl.ds(e0, GC)], GI[j], SEMI[j])
        pltpu.async_copy(sidx_hbm.at[pl.ds(e0, GC)], SI[j], SEMI[j])

    def wait_idx(j):
        d = pltpu.make_async_copy(gidx_hbm.at[pl.ds(0, GC)], GI[j], SEMI[j])
        d.wait()
        d.wait()

    for p in range(NPASS // 2):
        pass_id = c * (NPASS // 2) + p

        def zacc(i, _):
            pltpu.sync_copy(zbuf, acc.at[pl.ds(t0 + i * 512, 512)])
            return 0

        lax.fori_loop(0, oshare, zacc, 0)

        xp = x2_hbm.at[pass_id]
        x0 = s * (N // NTILES)
        pltpu.sync_copy(xp.at[pl.ds(x0, 512)], obuf)
        pltpu.sync_copy(obuf, xs.at[pl.ds(x0, 512)])
        pltpu.sync_copy(xp.at[pl.ds(x0 + 512, 113)], obuf.at[pl.ds(0, 113)])
        pltpu.sync_copy(obuf.at[pl.ds(0, 113)], xs.at[pl.ds(x0 + 512, 113)])
        plsc.subcore_barrier()

        stage_idx(0, 0)
        stage_idx(1, 1)
        stage_idx(2, 2)
        wait_idx(0)
        pltpu.async_copy(xs.at[GI[0]], RW[0], SEMG[0])

        def qblock(it, _):
            for jj in range(4):
                k = it * 4 + jj
                jn = (jj + 1) % 4
                jp = (jj + 3) % 4

                @pl.when(k + 3 < TROWS)
                def _():
                    stage_idx(k + 3, jp)

                @pl.when(k + 1 < TROWS)
                def _():
                    wait_idx(jn)

                    @pl.when(k >= 3)
                    def _():
                        pltpu.make_async_copy(RW[jn], acc.at[SI[jn]],
                                              SEMS[jn]).wait()

                    pltpu.async_copy(xs.at[GI[jn]], RW[jn], SEMG[jn])

                pltpu.make_async_copy(xp.at[pl.ds(0, GC)], RW[jj],
                                      SEMG[jj]).wait()
                pltpu.async_copy(RW[jj], acc.at[SI[jj]], SEMS[jj], add=True)
            return 0

        lax.fori_loop(0, TROWS // 4, qblock, 0)
        for jj in range(4):
            pltpu.make_async_copy(RW[jj], acc.at[SI[jj]], SEMS[jj]).wait()
        plsc.subcore_barrier()

        def ocp(i, _):
            pltpu.sync_copy(acc.at[pl.ds(t0 + i * 512, 512)], obuf)
            pltpu.sync_copy(obuf,
                            out_hbm.at[pl.ds(t0 + i * 512, 512),
                                       pl.ds(pass_id * CP, CP)])
            return 0

        lax.fori_loop(0, oshare, ocp, 0)
        plsc.subcore_barrier()


BN = 400


def _bf(a):
    return a.astype(jnp.bfloat16).astype(jnp.float32)


def _combine_body(act, s_ref, cnt_ref, h_ref,
                  basis_ref, comp_ref, root_ref, bias_ref, o_ref):
    hb = _bf(h_ref[...])
    acc = jnp.dot(hb, _bf(root_ref[...]),
                  preferred_element_type=jnp.float32)
    basis_b = _bf(basis_ref[...])
    for r in range(R):
        w_r = _bf(comp_ref[r, 0]) * basis_b[0]
        for b in range(1, NB):
            w_r = w_r + _bf(comp_ref[r, b]) * basis_b[b]
        mean_r = s_ref[r] / jnp.maximum(cnt_ref[:, r], 1.0)[:, None]
        acc = acc + jnp.dot(_bf(mean_r), _bf(w_r),
                            preferred_element_type=jnp.float32)
    acc = acc + bias_ref[...]
    if act == 0:
        res = jnp.maximum(acc, 0.0)
    else:
        mx = jnp.max(acc, axis=1, keepdims=True)
        e = jnp.exp(acc - mx)
        res = e / jnp.sum(e, axis=1, keepdims=True)
    o_ref[...] = res


def _combine(act, sp3, cnt2, h, basis, comp, root, bias):
    return pl.pallas_call(
        functools.partial(_combine_body, act),
        grid=(N // BN,),
        in_specs=[
            pl.BlockSpec((R, BN, D), lambda i: (0, i, 0)),
            pl.BlockSpec((BN, R), lambda i: (i, 0)),
            pl.BlockSpec((BN, D), lambda i: (i, 0)),
            pl.BlockSpec((NB, D, D), lambda i: (0, 0, 0)),
            pl.BlockSpec(memory_space=pltpu.SMEM),
            pl.BlockSpec((D, D), lambda i: (0, 0)),
            pl.BlockSpec((1, D), lambda i: (0, 0)),
        ],
        out_specs=pl.BlockSpec((BN, D), lambda i: (i, 0)),
        out_shape=jax.ShapeDtypeStruct((N, D), jnp.float32),
    )(sp3, cnt2, h, basis, comp, root, bias.reshape(1, D))


def kernel(x, edge_index, edge_type, basis0, comp0, root0, bias0, basis1,
           comp1, root1, bias1, basis2, comp2, root2, bias2):
    src = edge_index[0]
    dst = edge_index[1]
    sidx, gidx, cnt = _prep_kernel(src, dst, edge_type)
    cnt2 = cnt.reshape(N, R)
    h = x
    layers = [(basis0, comp0, root0, bias0, 0),
              (basis1, comp1, root1, bias1, 0),
              (basis2, comp2, root2, bias2, 1)]
    for basis, comp, root, bias, act in layers:
        x2 = jnp.transpose(h.reshape(N, NPASS, CP), (1, 0, 2))
        sums = _agg_kernel(x2, gidx, sidx)
        sp3 = sums.reshape(R, NP2, D)
        h = _combine(act, sp3, cnt2, h, basis, comp, root, bias)
    return h
